# baseline (speedup 1.0000x reference)
import functools

import jax
import jax.numpy as jnp
from jax import lax
from jax.experimental import pallas as pl
from jax.experimental.pallas import tpu as pltpu
from jax.experimental.pallas import tpu_sc as plsc

T = 2048
D = 2048
DFF = 1408
E = 8
TOPK = 2
NGROUP = 4
GSIZE = E // NGROUP
TOPK_GROUP = 2
NSHARED = 2

LANES = 128
TILE = 256
A = T * TOPK
M_TILES = A // TILE + E
PAD_ROWS = M_TILES * TILE
CB = 512
NCB = A // CB


def _router_body(x_ref, gw_ref, w_ref, pos_ref, te_ref):
    x = x_ref[...]
    gw = gw_ref[...]
    logits = lax.dot_general(x, gw, (((1,), (1,)), ((), ())),
                             preferred_element_type=jnp.float32)
    lane = lax.broadcasted_iota(jnp.int32, (T, LANES), 1)
    valid = lane < E
    lg = jnp.where(valid, logits, -1e30)
    mx = jnp.max(lg, axis=1, keepdims=True)
    ex = jnp.where(valid, jnp.exp(lg - mx), 0.0)
    p = ex / jnp.sum(ex, axis=1, keepdims=True)

    gs = [jnp.maximum(p[:, 2 * g:2 * g + 1], p[:, 2 * g + 1:2 * g + 2])
          for g in range(NGROUP)]
    sel = []
    for g in range(NGROUP):
        rank = None
        for g2 in range(NGROUP):
            beat = jnp.where(gs[g2] > gs[g], 1.0,
                             jnp.where((gs[g2] == gs[g]) & (g2 < g), 1.0, 0.0))
            rank = beat if rank is None else rank + beat
        sel.append(jnp.where(rank < TOPK_GROUP, 1.0, 0.0))
    gmask = jnp.zeros((T, LANES), jnp.float32)
    for g in range(NGROUP):
        gmask = jnp.where((lane // GSIZE) == g, sel[g], gmask)
    mp = p * gmask

    v1 = jnp.max(mp, axis=1, keepdims=True)
    i1 = jnp.min(jnp.where(mp == v1, lane, LANES), axis=1, keepdims=True)
    mp2 = jnp.where(lane == i1, -1.0, mp)
    v2 = jnp.max(mp2, axis=1, keepdims=True)
    i2 = jnp.min(jnp.where(mp2 == v2, lane, LANES), axis=1, keepdims=True)
    den = v1 + v2 + 1e-20
    w1 = v1 / den
    w2 = v2 / den
    w_ref[...] = jnp.where(lane == 0, w1, jnp.where(lane == 1, w2, 0.0))

    one0 = jnp.where(lane == i1, 1.0, 0.0)
    one1 = jnp.where(lane == i2, 1.0, 0.0)

    bl = lax.broadcasted_iota(jnp.int32, (CB, CB), 0)
    tri = jnp.where(bl >= lax.broadcasted_iota(jnp.int32, (CB, CB), 1),
                    1.0, 0.0)
    carry = jnp.zeros((1, LANES), jnp.float32)
    cums = []
    for b in range(NCB):
        r0 = b * CB
        if r0 + CB <= T:
            mb = one0[r0:r0 + CB]
        else:
            mb = one1[r0 - T:r0 - T + CB]
        cb = lax.dot_general(tri, mb, (((1,), (0,)), ((), ())),
                             preferred_element_type=jnp.float32) + carry
        carry = carry + jnp.sum(mb, axis=0, keepdims=True)
        cums.append(cb)

    counts_i = carry.astype(jnp.int32)
    pc = ((counts_i + (TILE - 1)) // TILE) * TILE
    pcf = pc.astype(jnp.float32)
    li = lax.broadcasted_iota(jnp.int32, (LANES, LANES), 0)
    tri_lt = jnp.where(li < lax.broadcasted_iota(jnp.int32, (LANES, LANES), 1),
                       1.0, 0.0)
    off = lax.dot_general(pcf, tri_lt, (((1,), (0,)), ((), ())),
                          preferred_element_type=jnp.float32)

    for b in range(NCB):
        r0 = b * CB
        mb = one0[r0:r0 + CB] if r0 + CB <= T else one1[r0 - T:r0 - T + CB]
        posb = jnp.sum((cums[b] - 1.0 + off) * mb, axis=1, keepdims=True)
        pos_ref[r0:r0 + CB, :] = posb.astype(jnp.int32)

    tl = lax.broadcasted_iota(jnp.int32, (1, LANES), 1).astype(jnp.float32)
    te = jnp.full((1, LANES), float(E - 1), jnp.float32)
    for e in range(E):
        st = off[:, e:e + 1] * (1.0 / TILE)
        nt = pcf[:, e:e + 1] * (1.0 / TILE)
        te = te - float(E - 1 - e) * jnp.where((tl >= st) & (tl < st + nt), 1.0, 0.0)
    te_ref[...] = te.astype(jnp.int32)


def _router(x, gw_pad):
    return pl.pallas_call(
        _router_body,
        out_shape=(
            jax.ShapeDtypeStruct((T, LANES), jnp.float32),
            jax.ShapeDtypeStruct((A, 1), jnp.int32),
            jax.ShapeDtypeStruct((1, LANES), jnp.int32),
        ),
    )(x, gw_pad)


_NC, _NS = 2, 16
_NW = _NC * _NS
_DCH = 16
_APW = A // _NW
_CCH = 16
_TPW = T // _NW


def _dispatch(x, pos):
    mesh = plsc.VectorSubcoreMesh(core_axis_name="c", subcore_axis_name="s")
    nch = _APW // _DCH

    @functools.partial(
        pl.kernel, mesh=mesh,
        out_type=jax.ShapeDtypeStruct((PAD_ROWS, D), jnp.float32),
        scratch_types=(
            [pltpu.VMEM((_APW,), jnp.int32)]
            + [pltpu.VMEM((_DCH,), jnp.int32) for _ in range(3)]
            + [pltpu.VMEM((_DCH, D), jnp.float32) for _ in range(3)]
            + [pltpu.SemaphoreType.DMA for _ in range(6)]
        ),
    )
    def k(x_hbm, pos_hbm, xs_hbm, idx_all,
          i0, i1, i2, r0, r1, r2, ls0, ls1, ls2, ss0, ss1, ss2):
        idxb, rows = [i0, i1, i2], [r0, r1, r2]
        ls, ss = [ls0, ls1, ls2], [ss0, ss1, ss2]
        wid = lax.axis_index("s") * _NC + lax.axis_index("c")
        base = wid * _APW
        t0 = base % T
        pltpu.sync_copy(pos_hbm.at[pl.ds(base, _APW)], idx_all)
        ld, sc = {}, {}
        for c in range(min(3, nch)):
            ld[c] = pltpu.async_copy(
                x_hbm.at[pl.ds(t0 + c * _DCH, _DCH)], rows[c % 3], ls[c % 3])
        for c in range(nch):
            b = c % 3
            ld[c].wait()
            for q in range(_DCH // 16):
                idxb[b][pl.ds(q * 16, 16)] = idx_all[pl.ds(c * _DCH + q * 16, 16)]
            sc[c] = pltpu.async_copy(rows[b], xs_hbm.at[idxb[b]], ss[b])
            if c + 3 < nch:
                sc[c].wait()
                ld[c + 3] = pltpu.async_copy(
                    x_hbm.at[pl.ds(t0 + (c + 3) * _DCH, _DCH)], rows[b], ls[b])
        for c in range(max(nch - 3, 0), nch):
            sc[c].wait()

    return k(x, pos)


def _silu(g):
    return g / (1.0 + jnp.exp(-g))


def _moe_h_body(te_ref, xs_ref, wgu_ref, h_ref):
    x = xs_ref[...].astype(jnp.bfloat16)
    g = lax.dot_general(x, wgu_ref[0, :DFF, :], (((1,), (1,)), ((), ())),
                        preferred_element_type=jnp.float32)
    u = lax.dot_general(x, wgu_ref[0, DFF:, :], (((1,), (1,)), ((), ())),
                        preferred_element_type=jnp.float32)
    h_ref[...] = (_silu(g) * u).astype(jnp.bfloat16)


def _moe_d_body(te_ref, h_ref, wd_ref, ys_ref):
    ys_ref[...] = lax.dot_general(h_ref[...], wd_ref[0],
                                  (((1,), (1,)), ((), ())),
                                  preferred_element_type=jnp.float32)


def _moe(te, xs, w_gate_up, w_down):
    h = pl.pallas_call(
        _moe_h_body,
        grid_spec=pltpu.PrefetchScalarGridSpec(
            num_scalar_prefetch=1,
            grid=(M_TILES,),
            in_specs=[
                pl.BlockSpec((TILE, D), lambda m, te: (m, 0)),
                pl.BlockSpec((1, 2 * DFF, D), lambda m, te: (te[m], 0, 0)),
            ],
            out_specs=pl.BlockSpec((TILE, DFF), lambda m, te: (m, 0)),
        ),
        out_shape=jax.ShapeDtypeStruct((PAD_ROWS, DFF), jnp.bfloat16),
    )(te, xs, w_gate_up)
    return pl.pallas_call(
        _moe_d_body,
        grid_spec=pltpu.PrefetchScalarGridSpec(
            num_scalar_prefetch=1,
            grid=(M_TILES,),
            in_specs=[
                pl.BlockSpec((TILE, DFF), lambda m, te: (m, 0)),
                pl.BlockSpec((1, D, DFF), lambda m, te: (te[m], 0, 0)),
            ],
            out_specs=pl.BlockSpec((TILE, D), lambda m, te: (m, 0)),
        ),
        out_shape=jax.ShapeDtypeStruct((PAD_ROWS, D), jnp.float32),
    )(te, h, w_down)


_CC = 8
_CN = _TPW // _CC


def _combine(ys, pos, w0, w1, shared):
    mesh = plsc.VectorSubcoreMesh(core_axis_name="c", subcore_axis_name="s")

    @functools.partial(
        pl.kernel, mesh=mesh,
        out_type=jax.ShapeDtypeStruct((T, D), jnp.float32),
        scratch_types=(
            [pltpu.VMEM((16,), jnp.int32) for _ in range(4)]
            + [pltpu.VMEM((_TPW + 16,), jnp.int32) for _ in range(2)]
            + [pltpu.VMEM((_TPW,), jnp.float32) for _ in range(2)]
            + [pltpu.VMEM((_CC, D), jnp.float32) for _ in range(6)]
            + [pltpu.SemaphoreType.DMA for _ in range(6)]
        ),
    )
    def k(ys_hbm, pos_hbm, w0_hbm, w1_hbm, sh_hbm, out_hbm,
          ia0, ib0, ia1, ib1, idx0_all, idx1_all, w0all, w1all,
          y0a, y0b, y1a, y1b, sh0, sh1,
          ga, gb, hs0, hs1, st0, st1):
        i0s, i1s = [ia0, ia1], [ib0, ib1]
        y0s, y1s, gsem = [y0a, y0b], [y1a, y1b], [ga, gb]
        shs, shsem, stsem = [sh0, sh1], [hs0, hs1], [st0, st1]
        wid = lax.axis_index("s") * _NC + lax.axis_index("c")
        base = wid * _TPW
        pltpu.sync_copy(pos_hbm.at[pl.ds(base, _TPW)], idx0_all.at[pl.ds(0, _TPW)])
        pltpu.sync_copy(pos_hbm.at[pl.ds(T + base, _TPW)], idx1_all.at[pl.ds(0, _TPW)])
        pltpu.sync_copy(w0_hbm.at[pl.ds(base, _TPW)], w0all)
        pltpu.sync_copy(w1_hbm.at[pl.ds(base, _TPW)], w1all)

        def issue_y(c):
            s = c % 2
            i0s[s][...] = idx0_all[pl.ds(c * _CC, 16)]
            i1s[s][...] = idx1_all[pl.ds(c * _CC, 16)]
            h0 = pltpu.async_copy(ys_hbm.at[i0s[s].at[pl.ds(0, _CC)]], y0s[s], gsem[s])
            h1 = pltpu.async_copy(ys_hbm.at[i1s[s].at[pl.ds(0, _CC)]], y1s[s], gsem[s])
            return h0, h1

        def issue_sh(c):
            k2 = c % 2
            return pltpu.async_copy(
                sh_hbm.at[pl.ds(base + c * _CC, _CC)], shs[k2], shsem[k2])

        yh = {0: issue_y(0), 1: issue_y(1)}
        shh = {0: issue_sh(0), 1: issue_sh(1)}
        sth = {}
        for c in range(_CN):
            s, k3 = c % 2, c % 2
            yh[c][0].wait()
            yh[c][1].wait()
            shh[c].wait()
            if c >= 1 and c + 1 < _CN:
                sth[c - 1].wait()
                shh[c + 1] = issue_sh(c + 1)
            wv0 = w0all[pl.ds((c // 2) * 16, 16)]
            wv1 = w1all[pl.ds((c // 2) * 16, 16)]
            for r in range(_CC):
                a = wv0[(c % 2) * _CC + r]
                b = wv1[(c % 2) * _CC + r]

                def col_loop(j, _, a=a, b=b, r=r, s=s, k3=k3):
                    for jj in range(4):
                        sl = pl.ds(j * 64 + jj * 16, 16)
                        shs[k3][r, sl] = (shs[k3][r, sl]
                                          + a * y0s[s][r, sl] + b * y1s[s][r, sl])
                    return 0

                lax.fori_loop(0, D // 64, col_loop, 0)
            sth[c] = pltpu.async_copy(
                shs[k3], out_hbm.at[pl.ds(base + c * _CC, _CC)], stsem[k3])
            if c + 2 < _CN:
                yh[c + 2] = issue_y(c + 2)
        for c in range(max(_CN - 2, 0), _CN):
            sth[c].wait()

    return k(ys, pos, w0, w1, shared)


_SN = 2 * NSHARED * DFF
_SH = NSHARED * DFF
_SC_CH = 2
_SCW = _SH // _SC_CH
_SM = 8
_SMT = T // _SM


def _shared_h_body(x_ref, sg_ref, su_ref, h_ref):
    x = x_ref[...].astype(jnp.bfloat16)
    g = lax.dot_general(x, sg_ref[...], (((1,), (1,)), ((), ())),
                        preferred_element_type=jnp.float32)
    u = lax.dot_general(x, su_ref[...], (((1,), (1,)), ((), ())),
                        preferred_element_type=jnp.float32)
    h_ref[...] = (_silu(g) * u).astype(jnp.bfloat16)


def _shared_d_body(h_ref, sd_ref, out_ref):
    c = pl.program_id(1)
    part = lax.dot_general(h_ref[...], sd_ref[...], (((1,), (1,)), ((), ())),
                           preferred_element_type=jnp.float32)

    @pl.when(c == 0)
    def _():
        out_ref[...] = part

    @pl.when(c > 0)
    def _():
        out_ref[...] = out_ref[...] + part


_SMB = 1024


def _shared_mlp(x, shared_gate_up, shared_down):
    h = pl.pallas_call(
        _shared_h_body,
        grid=(_SC_CH, _SM),
        in_specs=[
            pl.BlockSpec((_SMT, D), lambda c, m: (m, 0)),
            pl.BlockSpec((_SCW, D), lambda c, m: (c, 0)),
            pl.BlockSpec((_SCW, D), lambda c, m: (c + _SC_CH, 0)),
        ],
        out_specs=pl.BlockSpec((_SMT, _SCW), lambda c, m: (m, c)),
        out_shape=jax.ShapeDtypeStruct((T, _SH), jnp.bfloat16),
    )(x, shared_gate_up, shared_gate_up)
    return pl.pallas_call(
        _shared_d_body,
        grid=(T // _SMB, _SC_CH),
        in_specs=[
            pl.BlockSpec((_SMB, _SCW), lambda m, c: (m, c)),
            pl.BlockSpec((D, _SCW), lambda m, c: (0, c)),
        ],
        out_specs=pl.BlockSpec((_SMB, D), lambda m, c: (m, 0)),
        out_shape=jax.ShapeDtypeStruct((T, D), jnp.float32),
    )(h, shared_down)


def kernel(hidden_states, gate_w, w_gate_up, w_down, shared_gate_up, shared_down):
    x = hidden_states
    gw_pad = jnp.zeros((LANES, D), jnp.float32).at[:E].set(gate_w)
    wout, pos2, te2 = _router(x, gw_pad)
    pos = pos2[:, 0]
    te = te2[0]
    w0 = wout[:, 0]
    w1 = wout[:, 1]

    shared = _shared_mlp(x, shared_gate_up.astype(jnp.bfloat16),
                         shared_down.astype(jnp.bfloat16))
    xs = _dispatch(x, pos)
    ys = _moe(te[:M_TILES], xs, w_gate_up.astype(jnp.bfloat16),
              w_down.astype(jnp.bfloat16))
    return _combine(ys, pos, w0, w1, shared)


# speedup vs baseline: 1.2948x; 1.2948x over previous
import functools

import jax
import jax.numpy as jnp
from jax import lax
from jax.experimental import pallas as pl
from jax.experimental.pallas import tpu as pltpu
from jax.experimental.pallas import tpu_sc as plsc

T = 2048
D = 2048
DFF = 1408
E = 8
TOPK = 2
NGROUP = 4
GSIZE = E // NGROUP
TOPK_GROUP = 2
NSHARED = 2

LANES = 128
TILE = 256
A = T * TOPK
M_TILES = A // TILE + E
PAD_ROWS = M_TILES * TILE
CB = 512
NCB = A // CB


def _router_body(x_ref, gw_ref, w_ref, pos_ref, te_ref, act_ref):
    x = x_ref[...]
    gw = gw_ref[...]
    logits = lax.dot_general(x, gw, (((1,), (1,)), ((), ())),
                             preferred_element_type=jnp.float32)
    lane = lax.broadcasted_iota(jnp.int32, (T, LANES), 1)
    valid = lane < E
    lg = jnp.where(valid, logits, -1e30)
    mx = jnp.max(lg, axis=1, keepdims=True)
    ex = jnp.where(valid, jnp.exp(lg - mx), 0.0)
    p = ex / jnp.sum(ex, axis=1, keepdims=True)

    gs = [jnp.maximum(p[:, 2 * g:2 * g + 1], p[:, 2 * g + 1:2 * g + 2])
          for g in range(NGROUP)]
    sel = []
    for g in range(NGROUP):
        rank = None
        for g2 in range(NGROUP):
            beat = jnp.where(gs[g2] > gs[g], 1.0,
                             jnp.where((gs[g2] == gs[g]) & (g2 < g), 1.0, 0.0))
            rank = beat if rank is None else rank + beat
        sel.append(jnp.where(rank < TOPK_GROUP, 1.0, 0.0))
    gmask = jnp.zeros((T, LANES), jnp.float32)
    for g in range(NGROUP):
        gmask = jnp.where((lane // GSIZE) == g, sel[g], gmask)
    mp = p * gmask

    v1 = jnp.max(mp, axis=1, keepdims=True)
    i1 = jnp.min(jnp.where(mp == v1, lane, LANES), axis=1, keepdims=True)
    mp2 = jnp.where(lane == i1, -1.0, mp)
    v2 = jnp.max(mp2, axis=1, keepdims=True)
    i2 = jnp.min(jnp.where(mp2 == v2, lane, LANES), axis=1, keepdims=True)
    den = v1 + v2 + 1e-20
    w1 = v1 / den
    w2 = v2 / den
    w_ref[...] = jnp.where(lane == 0, w1, jnp.where(lane == 1, w2, 0.0))

    one0 = jnp.where(lane == i1, 1.0, 0.0)
    one1 = jnp.where(lane == i2, 1.0, 0.0)

    bl = lax.broadcasted_iota(jnp.int32, (CB, CB), 0)
    tri = jnp.where(bl >= lax.broadcasted_iota(jnp.int32, (CB, CB), 1),
                    1.0, 0.0)
    carry = jnp.zeros((1, LANES), jnp.float32)
    cums = []
    for b in range(NCB):
        r0 = b * CB
        if r0 + CB <= T:
            mb = one0[r0:r0 + CB]
        else:
            mb = one1[r0 - T:r0 - T + CB]
        cb = lax.dot_general(tri, mb, (((1,), (0,)), ((), ())),
                             preferred_element_type=jnp.float32) + carry
        carry = carry + jnp.sum(mb, axis=0, keepdims=True)
        cums.append(cb)

    counts_i = carry.astype(jnp.int32)
    pc = ((counts_i + (TILE - 1)) // TILE) * TILE
    pcf = pc.astype(jnp.float32)
    li = lax.broadcasted_iota(jnp.int32, (LANES, LANES), 0)
    tri_lt = jnp.where(li < lax.broadcasted_iota(jnp.int32, (LANES, LANES), 1),
                       1.0, 0.0)
    off = lax.dot_general(pcf, tri_lt, (((1,), (0,)), ((), ())),
                          preferred_element_type=jnp.float32)

    for b in range(NCB):
        r0 = b * CB
        mb = one0[r0:r0 + CB] if r0 + CB <= T else one1[r0 - T:r0 - T + CB]
        posb = jnp.sum((cums[b] - 1.0 + off) * mb, axis=1, keepdims=True)
        pos_ref[r0:r0 + CB, :] = posb.astype(jnp.int32)

    tl = lax.broadcasted_iota(jnp.int32, (1, LANES), 1).astype(jnp.float32)
    te = jnp.full((1, LANES), float(E - 1), jnp.float32)
    for e in range(E):
        st = off[:, e:e + 1] * (1.0 / TILE)
        nt = pcf[:, e:e + 1] * (1.0 / TILE)
        te = te - float(E - 1 - e) * jnp.where((tl >= st) & (tl < st + nt), 1.0, 0.0)
    te_ref[...] = te.astype(jnp.int32)
    tot = jnp.sum(pcf * (1.0 / TILE), axis=1, keepdims=True)
    act_ref[...] = jnp.where(tl < tot, 1, 0).astype(jnp.int32)


def _router(x, gw_pad):
    return pl.pallas_call(
        _router_body,
        out_shape=(
            jax.ShapeDtypeStruct((T, LANES), jnp.float32),
            jax.ShapeDtypeStruct((A, 1), jnp.int32),
            jax.ShapeDtypeStruct((1, LANES), jnp.int32),
            jax.ShapeDtypeStruct((1, LANES), jnp.int32),
        ),
    )(x, gw_pad)


_NC, _NS = 2, 16
_NW = _NC * _NS
_DCH = 16
_APW = A // _NW
_CCH = 16
_TPW = T // _NW


def _dispatch(x, pos):
    mesh = plsc.VectorSubcoreMesh(core_axis_name="c", subcore_axis_name="s")
    nch = _APW // _DCH

    @functools.partial(
        pl.kernel, mesh=mesh,
        out_type=jax.ShapeDtypeStruct((PAD_ROWS, D), jnp.float32),
        scratch_types=(
            [pltpu.VMEM((_APW,), jnp.int32)]
            + [pltpu.VMEM((_DCH,), jnp.int32) for _ in range(3)]
            + [pltpu.VMEM((_DCH, D), jnp.float32) for _ in range(3)]
            + [pltpu.SemaphoreType.DMA for _ in range(6)]
        ),
    )
    def k(x_hbm, pos_hbm, xs_hbm, idx_all,
          i0, i1, i2, r0, r1, r2, ls0, ls1, ls2, ss0, ss1, ss2):
        idxb, rows = [i0, i1, i2], [r0, r1, r2]
        ls, ss = [ls0, ls1, ls2], [ss0, ss1, ss2]
        wid = lax.axis_index("s") * _NC + lax.axis_index("c")
        base = wid * _APW
        t0 = base % T
        pltpu.sync_copy(pos_hbm.at[pl.ds(base, _APW)], idx_all)
        ld, sc = {}, {}
        for c in range(min(3, nch)):
            ld[c] = pltpu.async_copy(
                x_hbm.at[pl.ds(t0 + c * _DCH, _DCH)], rows[c % 3], ls[c % 3])
        for c in range(nch):
            b = c % 3
            ld[c].wait()
            for q in range(_DCH // 16):
                idxb[b][pl.ds(q * 16, 16)] = idx_all[pl.ds(c * _DCH + q * 16, 16)]
            sc[c] = pltpu.async_copy(rows[b], xs_hbm.at[idxb[b]], ss[b])
            if c + 3 < nch:
                sc[c].wait()
                ld[c + 3] = pltpu.async_copy(
                    x_hbm.at[pl.ds(t0 + (c + 3) * _DCH, _DCH)], rows[b], ls[b])
        for c in range(max(nch - 3, 0), nch):
            sc[c].wait()

    return k(x, pos)


def _silu(g):
    return g / (1.0 + jnp.exp(-g))


def _moe_h_body(te_ref, act_ref, xs_ref, wgu_ref, h_ref):
    @pl.when(act_ref[pl.program_id(0)] == 1)
    def _():
        x = xs_ref[...]
        g = lax.dot_general(x, wgu_ref[0, :DFF, :], (((1,), (1,)), ((), ())),
                            preferred_element_type=jnp.float32)
        u = lax.dot_general(x, wgu_ref[0, DFF:, :], (((1,), (1,)), ((), ())),
                            preferred_element_type=jnp.float32)
        h_ref[...] = _silu(g) * u


def _moe_d_body(te_ref, act_ref, h_ref, wd_ref, ys_ref):
    @pl.when(act_ref[pl.program_id(0)] == 1)
    def _():
        ys_ref[...] = lax.dot_general(h_ref[...], wd_ref[0],
                                      (((1,), (1,)), ((), ())),
                                      preferred_element_type=jnp.float32)


def _moe(te, act, xs, w_gate_up, w_down):
    h = pl.pallas_call(
        _moe_h_body,
        grid_spec=pltpu.PrefetchScalarGridSpec(
            num_scalar_prefetch=2,
            grid=(M_TILES,),
            in_specs=[
                pl.BlockSpec((TILE, D), lambda m, te, act: (m, 0)),
                pl.BlockSpec((1, 2 * DFF, D), lambda m, te, act: (te[m], 0, 0)),
            ],
            out_specs=pl.BlockSpec((TILE, DFF), lambda m, te, act: (m, 0)),
        ),
        out_shape=jax.ShapeDtypeStruct((PAD_ROWS, DFF), jnp.float32),
    )(te, act, xs, w_gate_up)
    return pl.pallas_call(
        _moe_d_body,
        grid_spec=pltpu.PrefetchScalarGridSpec(
            num_scalar_prefetch=2,
            grid=(M_TILES,),
            in_specs=[
                pl.BlockSpec((TILE, DFF), lambda m, te, act: (m, 0)),
                pl.BlockSpec((1, D, DFF), lambda m, te, act: (te[m], 0, 0)),
            ],
            out_specs=pl.BlockSpec((TILE, D), lambda m, te, act: (m, 0)),
        ),
        out_shape=jax.ShapeDtypeStruct((PAD_ROWS, D), jnp.float32),
    )(te, act, h, w_down)


_CC = 8
_CN = _TPW // _CC


def _combine(ys, pos, w0, w1, shared):
    mesh = plsc.VectorSubcoreMesh(core_axis_name="c", subcore_axis_name="s")

    @functools.partial(
        pl.kernel, mesh=mesh,
        out_type=jax.ShapeDtypeStruct((T, D), jnp.float32),
        scratch_types=(
            [pltpu.VMEM((16,), jnp.int32) for _ in range(4)]
            + [pltpu.VMEM((_TPW + 16,), jnp.int32) for _ in range(2)]
            + [pltpu.VMEM((_TPW,), jnp.float32) for _ in range(2)]
            + [pltpu.VMEM((_CC, D), jnp.float32) for _ in range(6)]
            + [pltpu.SemaphoreType.DMA for _ in range(6)]
        ),
    )
    def k(ys_hbm, pos_hbm, w0_hbm, w1_hbm, sh_hbm, out_hbm,
          ia0, ib0, ia1, ib1, idx0_all, idx1_all, w0all, w1all,
          y0a, y0b, y1a, y1b, sh0, sh1,
          ga, gb, hs0, hs1, st0, st1):
        i0s, i1s = [ia0, ia1], [ib0, ib1]
        y0s, y1s, gsem = [y0a, y0b], [y1a, y1b], [ga, gb]
        shs, shsem, stsem = [sh0, sh1], [hs0, hs1], [st0, st1]
        wid = lax.axis_index("s") * _NC + lax.axis_index("c")
        base = wid * _TPW
        pltpu.sync_copy(pos_hbm.at[pl.ds(base, _TPW)], idx0_all.at[pl.ds(0, _TPW)])
        pltpu.sync_copy(pos_hbm.at[pl.ds(T + base, _TPW)], idx1_all.at[pl.ds(0, _TPW)])
        pltpu.sync_copy(w0_hbm.at[pl.ds(base, _TPW)], w0all)
        pltpu.sync_copy(w1_hbm.at[pl.ds(base, _TPW)], w1all)

        def issue_y(c):
            s = c % 2
            i0s[s][...] = idx0_all[pl.ds(c * _CC, 16)]
            i1s[s][...] = idx1_all[pl.ds(c * _CC, 16)]
            h0 = pltpu.async_copy(ys_hbm.at[i0s[s].at[pl.ds(0, _CC)]], y0s[s], gsem[s])
            h1 = pltpu.async_copy(ys_hbm.at[i1s[s].at[pl.ds(0, _CC)]], y1s[s], gsem[s])
            return h0, h1

        def issue_sh(c):
            k2 = c % 2
            return pltpu.async_copy(
                sh_hbm.at[pl.ds(base + c * _CC, _CC)], shs[k2], shsem[k2])

        yh = {0: issue_y(0), 1: issue_y(1)}
        shh = {0: issue_sh(0), 1: issue_sh(1)}
        sth = {}
        for c in range(_CN):
            s, k3 = c % 2, c % 2
            yh[c][0].wait()
            yh[c][1].wait()
            shh[c].wait()
            if c >= 1 and c + 1 < _CN:
                sth[c - 1].wait()
                shh[c + 1] = issue_sh(c + 1)
            wv0 = w0all[pl.ds((c // 2) * 16, 16)]
            wv1 = w1all[pl.ds((c // 2) * 16, 16)]
            for r in range(_CC):
                a = wv0[(c % 2) * _CC + r]
                b = wv1[(c % 2) * _CC + r]

                def col_loop(j, _, a=a, b=b, r=r, s=s, k3=k3):
                    for jj in range(4):
                        sl = pl.ds(j * 64 + jj * 16, 16)
                        shs[k3][r, sl] = (shs[k3][r, sl]
                                          + a * y0s[s][r, sl] + b * y1s[s][r, sl])
                    return 0

                lax.fori_loop(0, D // 64, col_loop, 0)
            sth[c] = pltpu.async_copy(
                shs[k3], out_hbm.at[pl.ds(base + c * _CC, _CC)], stsem[k3])
            if c + 2 < _CN:
                yh[c + 2] = issue_y(c + 2)
        for c in range(max(_CN - 2, 0), _CN):
            sth[c].wait()

    return k(ys, pos, w0, w1, shared)


_SN = 2 * NSHARED * DFF
_SH = NSHARED * DFF
_SC_CH = 2
_SCW = _SH // _SC_CH
_SM = 8
_SMT = T // _SM


def _shared_h_body(x_ref, sg_ref, su_ref, h_ref):
    x = x_ref[...]
    g = lax.dot_general(x, sg_ref[...], (((1,), (1,)), ((), ())),
                        preferred_element_type=jnp.float32)
    u = lax.dot_general(x, su_ref[...], (((1,), (1,)), ((), ())),
                        preferred_element_type=jnp.float32)
    h_ref[...] = _silu(g) * u


def _shared_d_body(h_ref, sd_ref, out_ref):
    c = pl.program_id(1)
    part = lax.dot_general(h_ref[...], sd_ref[...], (((1,), (1,)), ((), ())),
                           preferred_element_type=jnp.float32)

    @pl.when(c == 0)
    def _():
        out_ref[...] = part

    @pl.when(c > 0)
    def _():
        out_ref[...] = out_ref[...] + part


_SMB = 1024


def _shared_mlp(x, shared_gate_up, shared_down):
    h = pl.pallas_call(
        _shared_h_body,
        grid=(_SC_CH, _SM),
        in_specs=[
            pl.BlockSpec((_SMT, D), lambda c, m: (m, 0)),
            pl.BlockSpec((_SCW, D), lambda c, m: (c, 0)),
            pl.BlockSpec((_SCW, D), lambda c, m: (c + _SC_CH, 0)),
        ],
        out_specs=pl.BlockSpec((_SMT, _SCW), lambda c, m: (m, c)),
        out_shape=jax.ShapeDtypeStruct((T, _SH), jnp.float32),
    )(x, shared_gate_up, shared_gate_up)
    return pl.pallas_call(
        _shared_d_body,
        grid=(T // _SMB, _SC_CH),
        in_specs=[
            pl.BlockSpec((_SMB, _SCW), lambda m, c: (m, c)),
            pl.BlockSpec((D, _SCW), lambda m, c: (0, c)),
        ],
        out_specs=pl.BlockSpec((_SMB, D), lambda m, c: (m, 0)),
        out_shape=jax.ShapeDtypeStruct((T, D), jnp.float32),
    )(h, shared_down)


def kernel(hidden_states, gate_w, w_gate_up, w_down, shared_gate_up, shared_down):
    x = hidden_states
    gw_pad = jnp.zeros((LANES, D), jnp.float32).at[:E].set(gate_w)
    wout, pos2, te2, act2 = _router(x, gw_pad)
    pos = pos2[:, 0]
    te = te2[0]
    w0 = wout[:, 0]
    w1 = wout[:, 1]

    shared = _shared_mlp(x, shared_gate_up, shared_down)
    xs = _dispatch(x, pos)
    ys = _moe(te[:M_TILES], act2[0][:M_TILES], xs, w_gate_up, w_down)
    return _combine(ys, pos, w0, w1, shared)


# speedup vs baseline: 1.3608x; 1.0510x over previous
import functools

import jax
import jax.numpy as jnp
from jax import lax
from jax.experimental import pallas as pl
from jax.experimental.pallas import tpu as pltpu
from jax.experimental.pallas import tpu_sc as plsc

T = 2048
D = 2048
DFF = 1408
E = 8
TOPK = 2
NGROUP = 4
GSIZE = E // NGROUP
TOPK_GROUP = 2
NSHARED = 2

LANES = 128
TILE = 256
A = T * TOPK
M_TILES = A // TILE + E
PAD_ROWS = M_TILES * TILE
CB = 512
NCB = A // CB


def _router_body(x_ref, gw_ref, w_ref, pos_ref, te_ref, act_ref):
    x = x_ref[...]
    gw = gw_ref[...]
    logits = lax.dot_general(x, gw, (((1,), (1,)), ((), ())),
                             preferred_element_type=jnp.float32)
    lane = lax.broadcasted_iota(jnp.int32, (T, LANES), 1)
    valid = lane < E
    lg = jnp.where(valid, logits, -1e30)
    mx = jnp.max(lg, axis=1, keepdims=True)
    ex = jnp.where(valid, jnp.exp(lg - mx), 0.0)
    p = ex / jnp.sum(ex, axis=1, keepdims=True)

    gs = [jnp.maximum(p[:, 2 * g:2 * g + 1], p[:, 2 * g + 1:2 * g + 2])
          for g in range(NGROUP)]
    sel = []
    for g in range(NGROUP):
        rank = None
        for g2 in range(NGROUP):
            beat = jnp.where(gs[g2] > gs[g], 1.0,
                             jnp.where((gs[g2] == gs[g]) & (g2 < g), 1.0, 0.0))
            rank = beat if rank is None else rank + beat
        sel.append(jnp.where(rank < TOPK_GROUP, 1.0, 0.0))
    gmask = jnp.zeros((T, LANES), jnp.float32)
    for g in range(NGROUP):
        gmask = jnp.where((lane // GSIZE) == g, sel[g], gmask)
    mp = p * gmask

    v1 = jnp.max(mp, axis=1, keepdims=True)
    i1 = jnp.min(jnp.where(mp == v1, lane, LANES), axis=1, keepdims=True)
    mp2 = jnp.where(lane == i1, -1.0, mp)
    v2 = jnp.max(mp2, axis=1, keepdims=True)
    i2 = jnp.min(jnp.where(mp2 == v2, lane, LANES), axis=1, keepdims=True)
    den = v1 + v2 + 1e-20
    w1 = v1 / den
    w2 = v2 / den
    w_ref[...] = jnp.where(lane == 0, w1, jnp.where(lane == 1, w2, 0.0))

    one0 = jnp.where(lane == i1, 1.0, 0.0)
    one1 = jnp.where(lane == i2, 1.0, 0.0)

    bl = lax.broadcasted_iota(jnp.int32, (CB, CB), 0)
    tri = jnp.where(bl >= lax.broadcasted_iota(jnp.int32, (CB, CB), 1),
                    1.0, 0.0)
    carry = jnp.zeros((1, LANES), jnp.float32)
    cums = []
    for b in range(NCB):
        r0 = b * CB
        if r0 + CB <= T:
            mb = one0[r0:r0 + CB]
        else:
            mb = one1[r0 - T:r0 - T + CB]
        cb = lax.dot_general(tri, mb, (((1,), (0,)), ((), ())),
                             preferred_element_type=jnp.float32) + carry
        carry = carry + jnp.sum(mb, axis=0, keepdims=True)
        cums.append(cb)

    counts_i = carry.astype(jnp.int32)
    pc = ((counts_i + (TILE - 1)) // TILE) * TILE
    pcf = pc.astype(jnp.float32)
    li = lax.broadcasted_iota(jnp.int32, (LANES, LANES), 0)
    tri_lt = jnp.where(li < lax.broadcasted_iota(jnp.int32, (LANES, LANES), 1),
                       1.0, 0.0)
    off = lax.dot_general(pcf, tri_lt, (((1,), (0,)), ((), ())),
                          preferred_element_type=jnp.float32)

    for b in range(NCB):
        r0 = b * CB
        mb = one0[r0:r0 + CB] if r0 + CB <= T else one1[r0 - T:r0 - T + CB]
        posb = jnp.sum((cums[b] - 1.0 + off) * mb, axis=1, keepdims=True)
        pos_ref[r0:r0 + CB, :] = posb.astype(jnp.int32)

    tl = lax.broadcasted_iota(jnp.int32, (1, LANES), 1).astype(jnp.float32)
    te = jnp.full((1, LANES), float(E - 1), jnp.float32)
    for e in range(E):
        st = off[:, e:e + 1] * (1.0 / TILE)
        nt = pcf[:, e:e + 1] * (1.0 / TILE)
        te = te - float(E - 1 - e) * jnp.where((tl >= st) & (tl < st + nt), 1.0, 0.0)
    te_ref[...] = te.astype(jnp.int32)
    tot = jnp.sum(pcf * (1.0 / TILE), axis=1, keepdims=True)
    act_ref[...] = jnp.where(tl < tot, 1, 0).astype(jnp.int32)


def _router(x, gw_pad):
    return pl.pallas_call(
        _router_body,
        out_shape=(
            jax.ShapeDtypeStruct((T, LANES), jnp.float32),
            jax.ShapeDtypeStruct((A, 1), jnp.int32),
            jax.ShapeDtypeStruct((1, LANES), jnp.int32),
            jax.ShapeDtypeStruct((1, LANES), jnp.int32),
        ),
    )(x, gw_pad)


_NC, _NS = 2, 16
_NW = _NC * _NS
_DCH = 16
_APW = A // _NW
_CCH = 16
_TPW = T // _NW


def _dispatch(x, pos):
    mesh = plsc.VectorSubcoreMesh(core_axis_name="c", subcore_axis_name="s")
    nch = _APW // _DCH

    @functools.partial(
        pl.kernel, mesh=mesh,
        out_type=jax.ShapeDtypeStruct((PAD_ROWS, D), jnp.float32),
        scratch_types=(
            [pltpu.VMEM((_APW,), jnp.int32)]
            + [pltpu.VMEM((_DCH,), jnp.int32) for _ in range(3)]
            + [pltpu.VMEM((_DCH, D), jnp.float32) for _ in range(3)]
            + [pltpu.SemaphoreType.DMA for _ in range(6)]
        ),
    )
    def k(x_hbm, pos_hbm, xs_hbm, idx_all,
          i0, i1, i2, r0, r1, r2, ls0, ls1, ls2, ss0, ss1, ss2):
        idxb, rows = [i0, i1, i2], [r0, r1, r2]
        ls, ss = [ls0, ls1, ls2], [ss0, ss1, ss2]
        wid = lax.axis_index("s") * _NC + lax.axis_index("c")
        base = wid * _APW
        t0 = base % T
        pltpu.sync_copy(pos_hbm.at[pl.ds(base, _APW)], idx_all)
        ld, sc = {}, {}
        for c in range(min(3, nch)):
            ld[c] = pltpu.async_copy(
                x_hbm.at[pl.ds(t0 + c * _DCH, _DCH)], rows[c % 3], ls[c % 3])
        for c in range(nch):
            b = c % 3
            ld[c].wait()
            for q in range(_DCH // 16):
                idxb[b][pl.ds(q * 16, 16)] = idx_all[pl.ds(c * _DCH + q * 16, 16)]
            sc[c] = pltpu.async_copy(rows[b], xs_hbm.at[idxb[b]], ss[b])
            if c + 3 < nch:
                sc[c].wait()
                ld[c + 3] = pltpu.async_copy(
                    x_hbm.at[pl.ds(t0 + (c + 3) * _DCH, _DCH)], rows[b], ls[b])
        for c in range(max(nch - 3, 0), nch):
            sc[c].wait()

    return k(x, pos)


def _silu(g):
    return g / (1.0 + jnp.exp(-g))


def _moe_h_body(te_ref, act_ref, xs_ref, wgu_ref, h_ref):
    @pl.when(act_ref[pl.program_id(0)] == 1)
    def _():
        x = xs_ref[...]
        g = lax.dot_general(x, wgu_ref[0, :DFF, :], (((1,), (1,)), ((), ())),
                            preferred_element_type=jnp.float32)
        u = lax.dot_general(x, wgu_ref[0, DFF:, :], (((1,), (1,)), ((), ())),
                            preferred_element_type=jnp.float32)
        h_ref[...] = _silu(g) * u


def _moe_d_body(te_ref, act_ref, h_ref, wd_ref, ys_ref):
    @pl.when(act_ref[pl.program_id(0)] == 1)
    def _():
        ys_ref[...] = lax.dot_general(h_ref[...], wd_ref[0],
                                      (((1,), (1,)), ((), ())),
                                      preferred_element_type=jnp.float32)


def _moe(te, act, xs, w_gate_up, w_down):
    h = pl.pallas_call(
        _moe_h_body,
        grid_spec=pltpu.PrefetchScalarGridSpec(
            num_scalar_prefetch=2,
            grid=(M_TILES,),
            in_specs=[
                pl.BlockSpec((TILE, D), lambda m, te, act: (m, 0)),
                pl.BlockSpec((1, 2 * DFF, D), lambda m, te, act: (te[m], 0, 0)),
            ],
            out_specs=pl.BlockSpec((TILE, DFF), lambda m, te, act: (m, 0)),
        ),
        out_shape=jax.ShapeDtypeStruct((PAD_ROWS, DFF), jnp.float32),
    )(te, act, xs, w_gate_up)
    return pl.pallas_call(
        _moe_d_body,
        grid_spec=pltpu.PrefetchScalarGridSpec(
            num_scalar_prefetch=2,
            grid=(M_TILES,),
            in_specs=[
                pl.BlockSpec((TILE, DFF), lambda m, te, act: (m, 0)),
                pl.BlockSpec((1, D, DFF), lambda m, te, act: (te[m], 0, 0)),
            ],
            out_specs=pl.BlockSpec((TILE, D), lambda m, te, act: (m, 0)),
        ),
        out_shape=jax.ShapeDtypeStruct((PAD_ROWS, D), jnp.float32),
    )(te, act, h, w_down)


_CC = 8
_CN = _TPW // _CC


def _combine(ys, pos, w0, w1, shared):
    mesh = plsc.VectorSubcoreMesh(core_axis_name="c", subcore_axis_name="s")

    @functools.partial(
        pl.kernel, mesh=mesh,
        out_type=jax.ShapeDtypeStruct((T, D), jnp.float32),
        scratch_types=(
            [pltpu.VMEM((16,), jnp.int32) for _ in range(4)]
            + [pltpu.VMEM((_TPW + 16,), jnp.int32) for _ in range(2)]
            + [pltpu.VMEM((_TPW,), jnp.float32) for _ in range(2)]
            + [pltpu.VMEM((_CC, D), jnp.float32) for _ in range(6)]
            + [pltpu.SemaphoreType.DMA for _ in range(6)]
        ),
    )
    def k(ys_hbm, pos_hbm, w0_hbm, w1_hbm, sh_hbm, out_hbm,
          ia0, ib0, ia1, ib1, idx0_all, idx1_all, w0all, w1all,
          y0a, y0b, y1a, y1b, sh0, sh1,
          ga, gb, hs0, hs1, st0, st1):
        i0s, i1s = [ia0, ia1], [ib0, ib1]
        y0s, y1s, gsem = [y0a, y0b], [y1a, y1b], [ga, gb]
        shs, shsem, stsem = [sh0, sh1], [hs0, hs1], [st0, st1]
        wid = lax.axis_index("s") * _NC + lax.axis_index("c")
        base = wid * _TPW
        pltpu.sync_copy(pos_hbm.at[pl.ds(base, _TPW)], idx0_all.at[pl.ds(0, _TPW)])
        pltpu.sync_copy(pos_hbm.at[pl.ds(T + base, _TPW)], idx1_all.at[pl.ds(0, _TPW)])
        pltpu.sync_copy(w0_hbm.at[pl.ds(base, _TPW)], w0all)
        pltpu.sync_copy(w1_hbm.at[pl.ds(base, _TPW)], w1all)

        def issue_y(c):
            s = c % 2
            i0s[s][...] = idx0_all[pl.ds(c * _CC, 16)]
            i1s[s][...] = idx1_all[pl.ds(c * _CC, 16)]
            h0 = pltpu.async_copy(ys_hbm.at[i0s[s].at[pl.ds(0, _CC)]], y0s[s], gsem[s])
            h1 = pltpu.async_copy(ys_hbm.at[i1s[s].at[pl.ds(0, _CC)]], y1s[s], gsem[s])
            return h0, h1

        def issue_sh(c):
            k2 = c % 2
            return pltpu.async_copy(
                sh_hbm.at[pl.ds(base + c * _CC, _CC)], shs[k2], shsem[k2])

        yh = {0: issue_y(0), 1: issue_y(1)}
        shh = {0: issue_sh(0), 1: issue_sh(1)}
        sth = {}
        for c in range(_CN):
            s, k3 = c % 2, c % 2
            yh[c][0].wait()
            yh[c][1].wait()
            shh[c].wait()
            if c >= 1 and c + 1 < _CN:
                sth[c - 1].wait()
                shh[c + 1] = issue_sh(c + 1)
            wv0 = w0all[pl.ds((c // 2) * 16, 16)]
            wv1 = w1all[pl.ds((c // 2) * 16, 16)]
            for r in range(_CC):
                a = wv0[(c % 2) * _CC + r]
                b = wv1[(c % 2) * _CC + r]

                def col_loop(j, _, a=a, b=b, r=r, s=s, k3=k3):
                    for jj in range(8):
                        sl = pl.ds(j * 128 + jj * 16, 16)
                        shs[k3][r, sl] = (shs[k3][r, sl]
                                          + a * y0s[s][r, sl] + b * y1s[s][r, sl])
                    return 0

                lax.fori_loop(0, D // 128, col_loop, 0)
            sth[c] = pltpu.async_copy(
                shs[k3], out_hbm.at[pl.ds(base + c * _CC, _CC)], stsem[k3])
            if c + 2 < _CN:
                yh[c + 2] = issue_y(c + 2)
        for c in range(max(_CN - 2, 0), _CN):
            sth[c].wait()

    return k(ys, pos, w0, w1, shared)


_SN = 2 * NSHARED * DFF
_SH = NSHARED * DFF
_SC_CH = 2
_SCW = _SH // _SC_CH
_SM = 8
_SMT = T // _SM


def _shared_h_body(x_ref, sg_ref, su_ref, h_ref):
    x = x_ref[...]
    g = lax.dot_general(x, sg_ref[...], (((1,), (1,)), ((), ())),
                        preferred_element_type=jnp.float32)
    u = lax.dot_general(x, su_ref[...], (((1,), (1,)), ((), ())),
                        preferred_element_type=jnp.float32)
    h_ref[...] = _silu(g) * u


def _shared_d_body(h_ref, sd_ref, out_ref):
    c = pl.program_id(1)
    part = lax.dot_general(h_ref[...], sd_ref[...], (((1,), (1,)), ((), ())),
                           preferred_element_type=jnp.float32)

    @pl.when(c == 0)
    def _():
        out_ref[...] = part

    @pl.when(c > 0)
    def _():
        out_ref[...] = out_ref[...] + part


_SMB = 1024


def _shared_mlp(x, shared_gate_up, shared_down):
    h = pl.pallas_call(
        _shared_h_body,
        grid=(_SC_CH, _SM),
        in_specs=[
            pl.BlockSpec((_SMT, D), lambda c, m: (m, 0)),
            pl.BlockSpec((_SCW, D), lambda c, m: (c, 0)),
            pl.BlockSpec((_SCW, D), lambda c, m: (c + _SC_CH, 0)),
        ],
        out_specs=pl.BlockSpec((_SMT, _SCW), lambda c, m: (m, c)),
        out_shape=jax.ShapeDtypeStruct((T, _SH), jnp.float32),
    )(x, shared_gate_up, shared_gate_up)
    return pl.pallas_call(
        _shared_d_body,
        grid=(T // _SMB, _SC_CH),
        in_specs=[
            pl.BlockSpec((_SMB, _SCW), lambda m, c: (m, c)),
            pl.BlockSpec((D, _SCW), lambda m, c: (0, c)),
        ],
        out_specs=pl.BlockSpec((_SMB, D), lambda m, c: (m, 0)),
        out_shape=jax.ShapeDtypeStruct((T, D), jnp.float32),
    )(h, shared_down)


def kernel(hidden_states, gate_w, w_gate_up, w_down, shared_gate_up, shared_down):
    x = hidden_states
    gw_pad = jnp.zeros((LANES, D), jnp.float32).at[:E].set(gate_w)
    wout, pos2, te2, act2 = _router(x, gw_pad)
    pos = pos2[:, 0]
    te = te2[0]
    w0 = wout[:, 0]
    w1 = wout[:, 1]

    shared = _shared_mlp(x, shared_gate_up, shared_down)
    xs = _dispatch(x, pos)
    ys = _moe(te[:M_TILES], act2[0][:M_TILES], xs, w_gate_up, w_down)
    return _combine(ys, pos, w0, w1, shared)
